# baseline (speedup 1.0000x reference)
import jax
from jax.experimental import pallas as pl
from jax.experimental.pallas import tpu as pltpu

_SCHED = (
    (0, 0, 256),
    (0, 256, 1792),
    (1, 0, 2048),
    (2, 0, 2048),
    (3, 0, 1792),
    (3, 1792, 256),
)
_WPIECES = ((0, 256), (256, 1792))


def _body(x_hbm, w_hbm, o_hbm, xb, wb, ob, sem_x, sem_w, sem_o):
    N = len(_SCHED)

    def x_in(c, start=True):
        b, r, n = _SCHED[c]
        cp = pltpu.make_async_copy(
            x_hbm.at[b, pl.ds(r, n), :], xb.at[c % 2, pl.ds(0, n), :],
            sem_x.at[c % 2])
        cp.start() if start else cp.wait()

    def o_out(c, start=True):
        b, r, n = _SCHED[c]
        cp = pltpu.make_async_copy(
            ob.at[c % 2, pl.ds(0, n), :], o_hbm.at[b, pl.ds(r, n), :],
            sem_o.at[c % 2])
        cp.start() if start else cp.wait()

    def w_in(p, start=True):
        r, n = _WPIECES[p]
        cp = pltpu.make_async_copy(
            w_hbm.at[pl.ds(r, n), :], wb.at[pl.ds(r, n), :], sem_w.at[p])
        cp.start() if start else cp.wait()

    w_in(0)
    x_in(0)
    w_in(1)
    x_in(1)

    for c in range(N):
        b, r, n = _SCHED[c]
        slot = c % 2
        x_in(c, start=False)
        if c < len(_WPIECES):
            w_in(c, start=False)
        if c >= 2:
            o_out(c - 2, start=False)
        ob[slot, :n] = xb[slot, :n] + wb[r:r + n]
        o_out(c)
        if c + 2 < N:
            x_in(c + 2)

    o_out(N - 2, start=False)
    o_out(N - 1, start=False)


def kernel(x, weight):
    B, S, D = x.shape
    w = weight[:S]
    return pl.pallas_call(
        _body,
        in_specs=[
            pl.BlockSpec(memory_space=pl.ANY),
            pl.BlockSpec(memory_space=pl.ANY),
        ],
        out_specs=pl.BlockSpec(memory_space=pl.ANY),
        out_shape=jax.ShapeDtypeStruct((B, S, D), x.dtype),
        scratch_shapes=[
            pltpu.VMEM((2, S, D), x.dtype),
            pltpu.VMEM((S, D), x.dtype),
            pltpu.VMEM((2, S, D), x.dtype),
            pltpu.SemaphoreType.DMA((2,)),
            pltpu.SemaphoreType.DMA((2,)),
            pltpu.SemaphoreType.DMA((2,)),
        ],
        compiler_params=pltpu.CompilerParams(vmem_limit_bytes=56 * 1024 * 1024),
    )(x, w)
